# Initial kernel scaffold; baseline (speedup 1.0000x reference)
import functools

import jax
import jax.numpy as jnp
from jax import lax
from jax.experimental import pallas as pl
from jax.experimental.pallas import tpu as pltpu
from jax.experimental.pallas import tpu_sc as plsc

N = 10000
E = 320000
IN_F = 128
OUT_F = 128
HEADS = 6

NP = 10240
B = 1024
NC = 2
NS = 16
K = 80
RPT = NP // NS
EPT_F = E // NS
EPT_C = E // (NC * NS)

_SELU_L = 1.0507009873554805
_SELU_A = 1.6732632423543772


def _prep_body(x_ref, wf_ref, msf_ref, mdf_ref, wc_ref, msc_ref, mdc_ref,
               hf_ref, asrct_ref, adstt_ref, hc_ref, ac_ref, ad_ref):
    x = x_ref[...]
    hf_ref[...] = jnp.dot(x, wf_ref[...], preferred_element_type=jnp.float32)
    dn = (((0,), (1,)), ((), ()))
    asrct_ref[...] = lax.dot_general(msf_ref[...], x, dn,
                                     preferred_element_type=jnp.float32)
    adstt_ref[...] = lax.dot_general(mdf_ref[...], x, dn,
                                     preferred_element_type=jnp.float32)
    hc_ref[...] = jnp.dot(x, wc_ref[...], preferred_element_type=jnp.float32)
    ac_ref[...] = jnp.dot(x, msc_ref[...], preferred_element_type=jnp.float32)
    ad_ref[...] = jnp.dot(x, mdc_ref[...], preferred_element_type=jnp.float32)


def _run_prep(data_p, w_feat, msf, mdf, wc16, msc16, mdc16):
    grid = (NP // B,)
    full = lambda shape: pl.BlockSpec(shape, lambda i: (0,) * len(shape))
    row = lambda w: pl.BlockSpec((B, w), lambda i: (i, 0))
    return pl.pallas_call(
        _prep_body,
        grid=grid,
        in_specs=[row(IN_F), full((IN_F, HEADS * OUT_F)), full((IN_F, HEADS)),
                  full((IN_F, HEADS)), full((IN_F, 16)), full((IN_F, 16)),
                  full((IN_F, 16))],
        out_specs=[row(HEADS * OUT_F),
                   pl.BlockSpec((HEADS, B), lambda i: (0, i)),
                   pl.BlockSpec((HEADS, B), lambda i: (0, i)),
                   row(16), row(16), row(16)],
        out_shape=[jax.ShapeDtypeStruct((NP, HEADS * OUT_F), jnp.float32),
                   jax.ShapeDtypeStruct((HEADS, NP), jnp.float32),
                   jax.ShapeDtypeStruct((HEADS, NP), jnp.float32),
                   jax.ShapeDtypeStruct((NP, 16), jnp.float32),
                   jax.ShapeDtypeStruct((NP, 16), jnp.float32),
                   jax.ShapeDtypeStruct((NP, 16), jnp.float32)],
    )(data_p, w_feat, msf, mdf, wc16, msc16, mdc16)


def _leaky_exp(a):
    return jnp.exp(jnp.where(a > 0, a, 0.2 * a))


def _sc_body(hf2, asrct, adstt, hc16, ac16, ad16, src_h, dst_h,
             numf, numc, denc,
             acc_f, den_f, numc_acc, denc_acc,
             a_s, a_d,
             sb0, db0, gb0, wb0, rows0,
             sb1, db1, gb1, wb1, rows1,
             csb, cdb, ca, cb2, wcb, crows,
             zrows, zden, zc, denv,
             gsem0, gsem1, ssem0, ssem1, dsem0, dsem1,
             cg0, cg1, cg2, cs0, cs1):
    cid = lax.axis_index("c")
    sid = lax.axis_index("s")
    zero16 = jnp.zeros((16,), jnp.float32)
    sb = (sb0, sb1)
    db = (db0, db1)
    gb = (gb0, gb1)
    wb = (wb0, wb1)
    rows = (rows0, rows1)
    gsem = (gsem0, gsem1)
    ssem = (ssem0, ssem1)
    dsem = (dsem0, dsem1)

    def _zrow(e, _):
        for j in range(8):
            zrows[e, pl.ds(16 * j, 16)] = zero16
        zc[e] = zero16
        return 0
    lax.fori_loop(0, K, _zrow, 0)
    for t in range(RPT // 16):
        zden[pl.ds(16 * t, 16)] = zero16
    r0 = sid * RPT
    for t in range(RPT // K):
        pltpu.sync_copy(zrows, acc_f.at[pl.ds(r0 + t * K, K)])
        pltpu.sync_copy(zc, numc_acc.at[pl.ds(r0 + t * K, K)])
        pltpu.sync_copy(zc, denc_acc.at[pl.ds(r0 + t * K, K)])
    pltpu.sync_copy(zden, den_f.at[pl.ds(r0, RPT)])
    plsc.subcore_barrier()

    hmap = lax.iota(jnp.int32, 16) // 2
    cbase = cid * (E // NC) + sid * EPT_C

    def _coord_chunk(k, _):
        off = cbase + k * K

        @pl.when(k > 0)
        def _():
            pltpu.make_async_copy(crows, numc_acc.at[cdb], cs0).wait()
            pltpu.make_async_copy(wcb, denc_acc.at[cdb], cs1).wait()

        pltpu.sync_copy(src_h.at[pl.ds(off, K)], csb)
        pltpu.sync_copy(dst_h.at[pl.ds(off, K)], cdb)
        pltpu.async_copy(hc16.at[csb], crows, cg0)
        pltpu.async_copy(ac16.at[csb], ca, cg1)
        pltpu.async_copy(ad16.at[cdb], cb2, cg2)
        pltpu.make_async_copy(hc16.at[csb], crows, cg0).wait()
        pltpu.make_async_copy(ac16.at[csb], ca, cg1).wait()
        pltpu.make_async_copy(ad16.at[cdb], cb2, cg2).wait()

        def _edge(e, _):
            w16 = _leaky_exp(ca[e] + cb2[e])
            wcb[e] = w16
            sv = plsc.load_gather(wcb, [jnp.full((16,), e, jnp.int32), hmap])
            crows[e] = crows[e] * sv
            return 0
        lax.fori_loop(0, K, _edge, 0)
        pltpu.async_copy(crows, numc_acc.at[cdb], cs0, add=True)
        pltpu.async_copy(wcb, denc_acc.at[cdb], cs1, add=True)
        return 0

    lax.fori_loop(0, EPT_C // K, _coord_chunk, 0)
    pltpu.make_async_copy(crows, numc_acc.at[cdb], cs0).wait()
    pltpu.make_async_copy(wcb, denc_acc.at[cdb], cs1).wait()
    plsc.subcore_barrier()
    pltpu.sync_copy(numc_acc.at[pl.ds(r0, RPT)], numc.at[cid, pl.ds(r0, RPT)])
    pltpu.sync_copy(denc_acc.at[pl.ds(r0, RPT)], denc.at[cid, pl.ds(r0, RPT)])

    ebase = sid * EPT_F
    for hsub in range(HEADS // NC):
        hh = cid * (HEADS // NC) + hsub
        pltpu.sync_copy(asrct.at[hh], a_s)
        pltpu.sync_copy(adstt.at[hh], a_d)

        def _pair(i, _):
            for b in range(2):
                c = 2 * i + b
                off = ebase + c * K

                @pl.when(i > 0)
                def _():
                    pltpu.make_async_copy(rows[b], acc_f.at[db[b]],
                                          ssem[b]).wait()
                    pltpu.make_async_copy(wb[b], den_f.at[db[b]],
                                          dsem[b]).wait()

                pltpu.sync_copy(src_h.at[pl.ds(off, K)], sb[b])
                pltpu.sync_copy(dst_h.at[pl.ds(off, K)], db[b])
                for t in range(K // 16):
                    sv = sb[b][pl.ds(16 * t, 16)]
                    gb[b][pl.ds(16 * t, 16)] = sv * HEADS + hh
                pltpu.async_copy(hf2.at[gb[b]], rows[b], gsem[b])
            for b in range(2):
                for t in range(K // 16):
                    sv = sb[b][pl.ds(16 * t, 16)]
                    dv = db[b][pl.ds(16 * t, 16)]
                    av = plsc.load_gather(a_s, [sv])
                    bv = plsc.load_gather(a_d, [dv])
                    wb[b][pl.ds(16 * t, 16)] = _leaky_exp(av + bv)
            for b in range(2):
                pltpu.make_async_copy(hf2.at[gb[b]], rows[b], gsem[b]).wait()

                def _scale(e, _):
                    w = wb[b][e]
                    for j in range(8):
                        rows[b][e, pl.ds(16 * j, 16)] = (
                            rows[b][e, pl.ds(16 * j, 16)] * w)
                    return 0
                lax.fori_loop(0, K, _scale, 0)
                pltpu.async_copy(rows[b], acc_f.at[db[b]], ssem[b], add=True)
                pltpu.async_copy(wb[b], den_f.at[db[b]], dsem[b], add=True)
            return 0

        lax.fori_loop(0, EPT_F // (2 * K), _pair, 0)
        for b in range(2):
            pltpu.make_async_copy(rows[b], acc_f.at[db[b]], ssem[b]).wait()
            pltpu.make_async_copy(wb[b], den_f.at[db[b]], dsem[b]).wait()
        plsc.subcore_barrier()

        pltpu.sync_copy(den_f.at[pl.ds(r0, RPT)], denv)
        for t in range(RPT // 16):
            v = denv[pl.ds(16 * t, 16)]
            denv[pl.ds(16 * t, 16)] = 1.0 / (v + 1e-16)
        for t in range(RPT // K):
            pltpu.sync_copy(acc_f.at[pl.ds(r0 + t * K, K)], rows0)

            def _norm(e, _):
                rsc = denv[t * K + e]
                for j in range(8):
                    rows0[e, pl.ds(16 * j, 16)] = (
                        rows0[e, pl.ds(16 * j, 16)] * rsc)
                return 0
            lax.fori_loop(0, K, _norm, 0)
            pltpu.sync_copy(rows0, numf.at[pl.ds(hh * NP + r0 + t * K, K)])
            pltpu.sync_copy(zrows, acc_f.at[pl.ds(r0 + t * K, K)])
        pltpu.sync_copy(zden, den_f.at[pl.ds(r0, RPT)])
        plsc.subcore_barrier()


def _run_sc(hf2, asrct, adstt, hc16, ac16, ad16, src, dst):
    mesh = plsc.VectorSubcoreMesh(core_axis_name="c", subcore_axis_name="s",
                                  num_cores=NC, num_subcores=NS)
    f32 = jnp.float32
    i32 = jnp.int32
    kern = pl.kernel(
        _sc_body,
        out_type=[jax.ShapeDtypeStruct((HEADS * NP, OUT_F), f32),
                  jax.ShapeDtypeStruct((NC, NP, 16), f32),
                  jax.ShapeDtypeStruct((NC, NP, 16), f32)],
        mesh=mesh,
        scratch_types=[
            pltpu.VMEM_SHARED((NP, OUT_F), f32),
            pltpu.VMEM_SHARED((NP,), f32),
            pltpu.VMEM_SHARED((NP, 16), f32),
            pltpu.VMEM_SHARED((NP, 16), f32),
            pltpu.VMEM((NP,), f32),
            pltpu.VMEM((NP,), f32),
            pltpu.VMEM((K,), i32), pltpu.VMEM((K,), i32),
            pltpu.VMEM((K,), i32), pltpu.VMEM((K,), f32),
            pltpu.VMEM((K, OUT_F), f32),
            pltpu.VMEM((K,), i32), pltpu.VMEM((K,), i32),
            pltpu.VMEM((K,), i32), pltpu.VMEM((K,), f32),
            pltpu.VMEM((K, OUT_F), f32),
            pltpu.VMEM((K,), i32), pltpu.VMEM((K,), i32),
            pltpu.VMEM((K, 16), f32), pltpu.VMEM((K, 16), f32),
            pltpu.VMEM((K, 16), f32), pltpu.VMEM((K, 16), f32),
            pltpu.VMEM((K, OUT_F), f32),
            pltpu.VMEM((RPT,), f32),
            pltpu.VMEM((K, 16), f32),
            pltpu.VMEM((RPT,), f32),
        ] + [pltpu.SemaphoreType.DMA] * 11,
    )
    return kern(hf2, asrct, adstt, hc16, ac16, ad16, src, dst)


def _final_body(numf_ref, numc_ref, denc_ref, data_ref, bf_ref, bc_ref,
                feat_ref, coord_ref):
    nf = numf_ref[...]
    f = (nf[0] + nf[1] + nf[2] + nf[3] + nf[4] + nf[5]) * (1.0 / HEADS)
    f = f + bf_ref[...]
    feat_ref[...] = _SELU_L * jnp.where(
        f > 0, f, _SELU_A * (jnp.exp(jnp.minimum(f, 0.0)) - 1.0))

    nc = numc_ref[0] + numc_ref[1]
    dc = denc_ref[0] + denc_ref[1]
    acc0 = jnp.zeros_like(nc[:, 0:1])
    acc1 = jnp.zeros_like(nc[:, 0:1])
    for h in range(HEADS):
        r = 1.0 / (dc[:, h:h + 1] + 1e-16)
        acc0 = acc0 + nc[:, 2 * h:2 * h + 1] * r
        acc1 = acc1 + nc[:, 2 * h + 1:2 * h + 2] * r
    c0 = acc0 * (1.0 / HEADS) + bc_ref[0, 0:1]
    c1 = acc1 * (1.0 / HEADS) + bc_ref[0, 1:2]
    d0 = data_ref[:, 0:1]
    d1 = data_ref[:, 1:2]
    c0 = jnp.where(d0 == 1.0, 1.0, c0)
    c0 = jnp.where(d0 == 0.0, 0.0, c0)
    c1 = jnp.where(d1 == 0.0, 0.0, c1)
    c1 = jnp.where(d1 == 1.0, 1.0, c1)
    pad = jnp.zeros((c0.shape[0], 14), jnp.float32)
    coord_ref[...] = jnp.concatenate([c0, c1, pad], axis=1)


def _run_final(numf3, numc, denc, data_p, bf, bc):
    grid = (NP // B,)
    return pl.pallas_call(
        _final_body,
        grid=grid,
        in_specs=[pl.BlockSpec((HEADS, B, OUT_F), lambda i: (0, i, 0)),
                  pl.BlockSpec((NC, B, 16), lambda i: (0, i, 0)),
                  pl.BlockSpec((NC, B, 16), lambda i: (0, i, 0)),
                  pl.BlockSpec((B, IN_F), lambda i: (i, 0)),
                  pl.BlockSpec((1, OUT_F), lambda i: (0, 0)),
                  pl.BlockSpec((1, 16), lambda i: (0, 0))],
        out_specs=[pl.BlockSpec((B, OUT_F), lambda i: (i, 0)),
                   pl.BlockSpec((B, 16), lambda i: (i, 0))],
        out_shape=[jax.ShapeDtypeStruct((NP, OUT_F), jnp.float32),
                   jax.ShapeDtypeStruct((NP, 16), jnp.float32)],
    )(numf3, numc, denc, data_p, bf, bc)


def kernel(data, edge_idx, W_feat, att_src_feat, att_dst_feat, bias_feat,
           W_coord, att_src_coord, att_dst_coord, bias_coord):
    data_p = jnp.pad(data, ((0, NP - N), (0, 0)))

    wf3 = W_feat.reshape(IN_F, HEADS, OUT_F)
    msf = jnp.einsum("khc,hc->kh", wf3, att_src_feat[0])
    mdf = jnp.einsum("khc,hc->kh", wf3, att_dst_feat[0])
    wc3 = W_coord.reshape(IN_F, HEADS, 2)
    msc16 = jnp.pad(jnp.einsum("khc,hc->kh", wc3, att_src_coord[0]),
                    ((0, 0), (0, 16 - HEADS)))
    mdc16 = jnp.pad(jnp.einsum("khc,hc->kh", wc3, att_dst_coord[0]),
                    ((0, 0), (0, 16 - HEADS)))
    wc16 = jnp.pad(W_coord, ((0, 0), (0, 16 - 2 * HEADS)))

    hf, asrct, adstt, hc16, ac16, ad16 = _run_prep(
        data_p, W_feat, msf, mdf, wc16, msc16, mdc16)

    hf2 = hf.reshape(NP * HEADS, OUT_F)
    src = edge_idx[0]
    dst = edge_idx[1]
    numf, numc, denc = _run_sc(hf2, asrct, adstt, hc16, ac16, ad16, src, dst)

    numf3 = numf.reshape(HEADS, NP, OUT_F)
    bf = bias_feat.reshape(1, OUT_F)
    bc = jnp.pad(bias_coord, (0, 14)).reshape(1, 16)
    feat, coord16 = _run_final(numf3, numc, denc, data_p, bf, bc)

    return (coord16[:N, :2], feat[:N])


# speedup vs baseline: 31.9309x; 31.9309x over previous
import functools

import jax
import jax.numpy as jnp
from jax import lax
from jax.experimental import pallas as pl
from jax.experimental.pallas import tpu as pltpu
from jax.experimental.pallas import tpu_sc as plsc

N = 10000
E = 320000
IN_F = 128
OUT_F = 128
HEADS = 6

NP = 10240
B = 1024
NC = 2
NS = 16
K = 80
RPT = NP // NS
EPT_F = E // NS
EPT_C = E // (NC * NS)

_SELU_L = 1.0507009873554805
_SELU_A = 1.6732632423543772


def _prep_body(x_ref, wf_ref, msf_ref, mdf_ref, wc_ref, msc_ref, mdc_ref,
               hf_ref, af_ref, adf_ref, hc_ref, ac_ref, ad_ref):
    x = x_ref[...]
    hf_ref[...] = jnp.dot(x, wf_ref[...], preferred_element_type=jnp.float32)
    af_ref[...] = jnp.dot(x, msf_ref[...], preferred_element_type=jnp.float32)
    adf_ref[...] = jnp.dot(x, mdf_ref[...], preferred_element_type=jnp.float32)
    hc_ref[...] = jnp.dot(x, wc_ref[...], preferred_element_type=jnp.float32)
    ac_ref[...] = jnp.dot(x, msc_ref[...], preferred_element_type=jnp.float32)
    ad_ref[...] = jnp.dot(x, mdc_ref[...], preferred_element_type=jnp.float32)


def _run_prep(data_p, w_feat, msf16, mdf16, wc16, msc16, mdc16):
    grid = (NP // B,)
    full = lambda shape: pl.BlockSpec(shape, lambda i: (0,) * len(shape))
    row = lambda w: pl.BlockSpec((B, w), lambda i: (i, 0))
    n16 = jax.ShapeDtypeStruct((NP, 16), jnp.float32)
    return pl.pallas_call(
        _prep_body,
        grid=grid,
        in_specs=[row(IN_F), full((IN_F, HEADS * OUT_F)), full((IN_F, 16)),
                  full((IN_F, 16)), full((IN_F, 16)), full((IN_F, 16)),
                  full((IN_F, 16))],
        out_specs=[row(HEADS * OUT_F), row(16), row(16),
                   row(16), row(16), row(16)],
        out_shape=[jax.ShapeDtypeStruct((NP, HEADS * OUT_F), jnp.float32),
                   n16, n16, n16, n16, n16],
    )(data_p, w_feat, msf16, mdf16, wc16, msc16, mdc16)


def _leaky_exp(a):
    return jnp.exp(jnp.where(a > 0, a, 0.2 * a))


def _sc_body(hf2, af16, adf16, hc16, ac16, ad16, src_h, dst_h,
             numf, numc,
             acc_f, den_f,
             sb0, db0, gb0, wb0, rows0,
             sb1, db1, gb1, wb1, rows1,
             ca0, ca1, cb0, cb1, cr0, cr1,
             w16b0, w16b1, z16,
             gsem0, gsem1, ssem0, ssem1, dsem0, dsem1,
             asem0, asem1, bsem0, bsem1):
    cid = lax.axis_index("c")
    sid = lax.axis_index("s")
    zero16 = jnp.zeros((16,), jnp.float32)
    zero16i = jnp.zeros((16,), jnp.int32)
    sb = (sb0, sb1)
    db = (db0, db1)
    gb = (gb0, gb1)
    wb = (wb0, wb1)
    rows = (rows0, rows1)
    ca = (ca0, ca1)
    cb = (cb0, cb1)
    cr = (cr0, cr1)
    gsem = (gsem0, gsem1)
    ssem = (ssem0, ssem1)
    dsem = (dsem0, dsem1)
    asem = (asem0, asem1)
    bsem = (bsem0, bsem1)
    w16b = (w16b0, w16b1)

    def _zero_rows(ref):
        def _zrow(e, _):
            for j in range(8):
                ref[e, pl.ds(16 * j, 16)] = zero16
            return 0
        lax.fori_loop(0, K, _zrow, 0)

    def _zero_16w(ref):
        def _zrow(e, _):
            ref[e] = zero16
            return 0
        lax.fori_loop(0, K, _zrow, 0)

    _zero_rows(rows0)
    _zero_rows(rows1)
    _zero_16w(w16b0)
    _zero_16w(w16b1)
    _zero_16w(z16)
    r0 = sid * RPT
    for t in range(RPT // K):
        pltpu.sync_copy(rows1, acc_f.at[pl.ds(r0 + t * K, K)])
        pltpu.sync_copy(z16, den_f.at[pl.ds(r0 + t * K, K)])
    plsc.subcore_barrier()

    cbase = cid * (E // NC) + sid * EPT_C

    def _cchunk(k, _):
        off = cbase + k * K

        @pl.when(k > 0)
        def _():
            pltpu.make_async_copy(rows0, acc_f.at[db0], ssem0).wait()

        pltpu.sync_copy(src_h.at[pl.ds(off, K)], sb0)
        pltpu.sync_copy(dst_h.at[pl.ds(off, K)], db0)
        pltpu.async_copy(hc16.at[sb0], cr0, gsem0)
        pltpu.async_copy(ac16.at[sb0], ca0, asem0)
        pltpu.async_copy(ad16.at[db0], cb0, bsem0)
        pltpu.make_async_copy(hc16.at[sb0], cr0, gsem0).wait()
        pltpu.make_async_copy(ac16.at[sb0], ca0, asem0).wait()
        pltpu.make_async_copy(ad16.at[db0], cb0, bsem0).wait()

        def _edge(e, _):
            w16 = _leaky_exp(ca0[e] + cb0[e])
            rows0[e, pl.ds(16, 16)] = w16
            rows0[e, pl.ds(0, 16)] = cr0[e] * w16
            return 0
        lax.fori_loop(0, K, _edge, 0)
        pltpu.async_copy(rows0, acc_f.at[db0], ssem0, add=True)
        return 0

    lax.fori_loop(0, EPT_C // K, _cchunk, 0)
    pltpu.make_async_copy(rows0, acc_f.at[db0], ssem0).wait()
    plsc.subcore_barrier()
    for t in range(RPT // K):
        pltpu.sync_copy(acc_f.at[pl.ds(r0 + t * K, K)], rows0)
        pltpu.sync_copy(rows0, numc.at[pl.ds(cid * NP + r0 + t * K, K)])
        pltpu.sync_copy(rows1, acc_f.at[pl.ds(r0 + t * K, K)])
    plsc.subcore_barrier()

    ebase = sid * EPT_F
    lane16 = lax.iota(jnp.int32, 16)

    def _head_pass(hsub, _):
        hh = cid * (HEADS // NC) + hsub
        hh16 = jnp.full((16,), hh, jnp.int32)

        def _pair(i, _):
            for b in range(2):
                c = 2 * i + b
                off = ebase + c * K

                @pl.when(i > 0)
                def _():
                    pltpu.make_async_copy(rows[b], acc_f.at[db[b]],
                                          ssem[b]).wait()
                    pltpu.make_async_copy(w16b[b], den_f.at[db[b]],
                                          dsem[b]).wait()

                pltpu.sync_copy(src_h.at[pl.ds(off, K)], sb[b])
                pltpu.sync_copy(dst_h.at[pl.ds(off, K)], db[b])
                for t in range(K // 16):
                    sv = sb[b][pl.ds(16 * t, 16)]
                    gb[b][pl.ds(16 * t, 16)] = sv * HEADS + hh
                pltpu.async_copy(hf2.at[gb[b]], rows[b], gsem[b])
                pltpu.async_copy(af16.at[sb[b]], ca[b], asem[b])
                pltpu.async_copy(adf16.at[db[b]], cb[b], bsem[b])
            for b in range(2):
                pltpu.make_async_copy(af16.at[sb[b]], ca[b], asem[b]).wait()
                pltpu.make_async_copy(adf16.at[db[b]], cb[b], bsem[b]).wait()
                for t in range(K // 16):
                    rowv = 16 * t + lane16
                    av = plsc.load_gather(ca[b], [rowv, hh16])
                    bv = plsc.load_gather(cb[b], [rowv, hh16])
                    wv = _leaky_exp(av + bv)
                    wb[b][pl.ds(16 * t, 16)] = wv
                    plsc.store_scatter(w16b[b], [rowv, zero16i], wv)
            for b in range(2):
                pltpu.make_async_copy(hf2.at[gb[b]], rows[b], gsem[b]).wait()

                def _scale(t, _):
                    wv = wb[b][pl.ds(16 * t, 16)]
                    for e16 in range(16):
                        w = wv[e16]
                        r = 16 * t + e16
                        for j in range(8):
                            rows[b][r, pl.ds(16 * j, 16)] = (
                                rows[b][r, pl.ds(16 * j, 16)] * w)
                    return 0
                lax.fori_loop(0, K // 16, _scale, 0)
                pltpu.async_copy(rows[b], acc_f.at[db[b]], ssem[b], add=True)
                pltpu.async_copy(w16b[b], den_f.at[db[b]], dsem[b], add=True)
            return 0

        lax.fori_loop(0, EPT_F // (2 * K), _pair, 0)
        for b in range(2):
            pltpu.make_async_copy(rows[b], acc_f.at[db[b]], ssem[b]).wait()
            pltpu.make_async_copy(w16b[b], den_f.at[db[b]], dsem[b]).wait()
        plsc.subcore_barrier()

        _zero_rows(rows1)
        for t in range(RPT // K):
            pltpu.sync_copy(acc_f.at[pl.ds(r0 + t * K, K)], rows0)
            pltpu.sync_copy(den_f.at[pl.ds(r0 + t * K, K)], ca0)

            def _norm(g, _):
                rowv = 16 * g + lane16
                dv = plsc.load_gather(ca0, [rowv, zero16i])
                rv = 1.0 / (dv + 1e-16)
                for e16 in range(16):
                    rsc = rv[e16]
                    r = 16 * g + e16
                    for j in range(8):
                        rows0[r, pl.ds(16 * j, 16)] = (
                            rows0[r, pl.ds(16 * j, 16)] * rsc)
                return 0
            lax.fori_loop(0, K // 16, _norm, 0)
            pltpu.sync_copy(rows0, numf.at[pl.ds(hh * NP + r0 + t * K, K)])
            pltpu.sync_copy(rows1, acc_f.at[pl.ds(r0 + t * K, K)])
            pltpu.sync_copy(z16, den_f.at[pl.ds(r0 + t * K, K)])
        plsc.subcore_barrier()
        return 0

    lax.fori_loop(0, HEADS // NC, _head_pass, 0)


def _run_sc(hf2, af16, adf16, hc16, ac16, ad16, src, dst):
    mesh = plsc.VectorSubcoreMesh(core_axis_name="c", subcore_axis_name="s",
                                  num_cores=NC, num_subcores=NS)
    f32 = jnp.float32
    i32 = jnp.int32
    kern = pl.kernel(
        _sc_body,
        out_type=[jax.ShapeDtypeStruct((HEADS * NP, OUT_F), f32),
                  jax.ShapeDtypeStruct((NC * NP, OUT_F), f32)],
        mesh=mesh,
        scratch_types=[
            pltpu.VMEM_SHARED((NP, OUT_F), f32),
            pltpu.VMEM_SHARED((NP, 16), f32),
            pltpu.VMEM((K,), i32), pltpu.VMEM((K,), i32),
            pltpu.VMEM((K,), i32), pltpu.VMEM((K,), f32),
            pltpu.VMEM((K, OUT_F), f32),
            pltpu.VMEM((K,), i32), pltpu.VMEM((K,), i32),
            pltpu.VMEM((K,), i32), pltpu.VMEM((K,), f32),
            pltpu.VMEM((K, OUT_F), f32),
            pltpu.VMEM((K, 16), f32), pltpu.VMEM((K, 16), f32),
            pltpu.VMEM((K, 16), f32), pltpu.VMEM((K, 16), f32),
            pltpu.VMEM((K, 16), f32), pltpu.VMEM((K, 16), f32),
            pltpu.VMEM((K, 16), f32), pltpu.VMEM((K, 16), f32),
            pltpu.VMEM((K, 16), f32),
        ] + [pltpu.SemaphoreType.DMA] * 10,
        compiler_params=pltpu.CompilerParams(needs_layout_passes=False,
                                             use_tc_tiling_on_sc=False),
    )
    return kern(hf2, af16, adf16, hc16, ac16, ad16, src, dst)


def _final_body(numf_ref, numc_ref, data_ref, bf_ref, bc_ref,
                feat_ref, coord_ref):
    nf = numf_ref[...]
    f = (nf[0] + nf[1] + nf[2] + nf[3] + nf[4] + nf[5]) * (1.0 / HEADS)
    f = f + bf_ref[...]
    feat_ref[...] = _SELU_L * jnp.where(
        f > 0, f, _SELU_A * (jnp.exp(jnp.minimum(f, 0.0)) - 1.0))

    nc = numc_ref[0] + numc_ref[1]
    acc0 = jnp.zeros_like(nc[:, 0:1])
    acc1 = jnp.zeros_like(nc[:, 0:1])
    for h in range(HEADS):
        r = 1.0 / (nc[:, 16 + h:17 + h] + 1e-16)
        acc0 = acc0 + nc[:, h:h + 1] * r
        acc1 = acc1 + nc[:, 8 + h:9 + h] * r
    c0 = acc0 * (1.0 / HEADS) + bc_ref[0, 0:1]
    c1 = acc1 * (1.0 / HEADS) + bc_ref[0, 1:2]
    d0 = data_ref[:, 0:1]
    d1 = data_ref[:, 1:2]
    c0 = jnp.where(d0 == 1.0, 1.0, c0)
    c0 = jnp.where(d0 == 0.0, 0.0, c0)
    c1 = jnp.where(d1 == 0.0, 0.0, c1)
    c1 = jnp.where(d1 == 1.0, 1.0, c1)
    pad = jnp.zeros((c0.shape[0], 14), jnp.float32)
    coord_ref[...] = jnp.concatenate([c0, c1, pad], axis=1)


def _run_final(numf3, numc3, data_p, bf, bc):
    grid = (NP // B,)
    return pl.pallas_call(
        _final_body,
        grid=grid,
        in_specs=[pl.BlockSpec((HEADS, B, OUT_F), lambda i: (0, i, 0)),
                  pl.BlockSpec((NC, B, OUT_F), lambda i: (0, i, 0)),
                  pl.BlockSpec((B, IN_F), lambda i: (i, 0)),
                  pl.BlockSpec((1, OUT_F), lambda i: (0, 0)),
                  pl.BlockSpec((1, 16), lambda i: (0, 0))],
        out_specs=[pl.BlockSpec((B, OUT_F), lambda i: (i, 0)),
                   pl.BlockSpec((B, 16), lambda i: (i, 0))],
        out_shape=[jax.ShapeDtypeStruct((NP, OUT_F), jnp.float32),
                   jax.ShapeDtypeStruct((NP, 16), jnp.float32)],
    )(numf3, numc3, data_p, bf, bc)


def kernel(data, edge_idx, W_feat, att_src_feat, att_dst_feat, bias_feat,
           W_coord, att_src_coord, att_dst_coord, bias_coord):
    data_p = jnp.pad(data, ((0, NP - N), (0, 0)))

    wf3 = W_feat.reshape(IN_F, HEADS, OUT_F)
    msf16 = jnp.pad(jnp.einsum("khc,hc->kh", wf3, att_src_feat[0]),
                    ((0, 0), (0, 16 - HEADS)))
    mdf16 = jnp.pad(jnp.einsum("khc,hc->kh", wf3, att_dst_feat[0]),
                    ((0, 0), (0, 16 - HEADS)))
    wc3 = W_coord.reshape(IN_F, HEADS, 2)
    msc = jnp.einsum("khc,hc->kh", wc3, att_src_coord[0])
    mdc = jnp.einsum("khc,hc->kh", wc3, att_dst_coord[0])
    pad2 = jnp.zeros((IN_F, 2), jnp.float32)
    msc16 = jnp.concatenate([msc, pad2, msc, pad2], axis=1)
    mdc16 = jnp.concatenate([mdc, pad2, mdc, pad2], axis=1)
    wc16 = jnp.concatenate([wc3[:, :, 0], pad2, wc3[:, :, 1], pad2], axis=1)

    hf, af16, adf16, hc16, ac16, ad16 = _run_prep(
        data_p, W_feat, msf16, mdf16, wc16, msc16, mdc16)

    hf2 = hf.reshape(NP * HEADS, OUT_F)
    src = edge_idx[0]
    dst = edge_idx[1]
    numf, numc = _run_sc(hf2, af16, adf16, hc16, ac16, ad16, src, dst)

    numf3 = numf.reshape(HEADS, NP, OUT_F)
    numc3 = numc.reshape(NC, NP, OUT_F)
    bf = bias_feat.reshape(1, OUT_F)
    bc = jnp.pad(bias_coord, (0, 14)).reshape(1, 16)
    feat, coord16 = _run_final(numf3, numc3, data_p, bf, bc)

    return (coord16[:N, :2], feat[:N])
